# baseline (speedup 1.0000x reference)
import jax
import jax.numpy as jnp
import numpy as np
from jax.experimental import pallas as pl
from jax.experimental.pallas import tpu as pltpu

_N = 1024
_IN = 256
_HID = 64
_HEADS = 4
_H4 = _HEADS * _HID
_LAT = 64


def _fused_kernel(x_ref, adj_ref,
                  w1_ref, as1_ref, ad1_ref, b1_ref,
                  w2_ref, as2_ref, ad2_ref, b2_ref,
                  fm1_ref, fmb1_ref, fm2_ref, fmb2_ref,
                  fl1_ref, flb1_ref, fl2_ref, flb2_ref,
                  atw_ref, atb_ref, eps_ref, dw_ref, db_ref,
                  rec_ref, mean_ref, logvar_ref,
                  adj_vmem, adj_sem):
    adj_cp = pltpu.make_async_copy(adj_ref, adj_vmem, adj_sem)
    adj_cp.start()
    _HX = _HID + 1
    lane = jax.lax.broadcasted_iota(jnp.int32, (1, _HEADS * _HX), 1)
    ones_row = jnp.where(lane % _HX == _HID, 1.0, 0.0).astype(jnp.float32)
    zc = jnp.zeros((_H4, 1), dtype=jnp.float32)

    def gat(inp, w_ref, asw_ref, adw_ref, b_ref, get_adj):
        w = w_ref[...]
        w_aug = jnp.concatenate(
            [t for k in range(_HEADS)
             for t in (w[:, _HID * k : _HID * (k + 1)], zc)], axis=1)
        h_aug = (jnp.dot(inp, w_aug, preferred_element_type=jnp.float32)
                 + ones_row)
        vecs = []
        for k in range(_HEADS):
            hk = h_aug[:, _HX * k : _HX * k + _HID]
            hx = h_aug[:, _HX * k : _HX * (k + 1)]
            a = jax.lax.dot_general(
                hk, asw_ref[k : k + 1, :], (((1,), (1,)), ((), ())),
                preferred_element_type=jnp.float32)
            bT = jax.lax.dot_general(
                adw_ref[k : k + 1, :], hk, (((1,), (1,)), ((), ())),
                preferred_element_type=jnp.float32)
            c1 = jnp.max(a)
            c2 = jnp.max(bT)
            half = 0.4 * (c1 + c2)
            ea = jnp.exp(a - c1)
            ebT = jnp.exp(bT - c2)
            ea2 = jnp.exp(0.2 * a - (0.2 * c1 + half))
            eb2T = jnp.exp(0.2 * bT - (0.2 * c2 + half))
            vecs.append((hx, ea, ebT, ea2, eb2T))
        adjv = get_adj()
        outs = []
        for hx, ea, ebT, ea2, eb2T in vecs:
            p = (adjv * jnp.maximum(ea * ebT, ea2 * eb2T)).astype(jnp.bfloat16)
            hxh = hx.astype(jnp.bfloat16)
            hxl = (hx - hxh.astype(jnp.float32)).astype(jnp.bfloat16)
            hx2 = jnp.concatenate([hxh, hxl], axis=1)
            os2 = jax.lax.dot_general(
                p, hx2, (((0,), (0,)), ((), ())),
                preferred_element_type=jnp.float32)
            os_ = os2[:, :_HX] + os2[:, _HX:]
            outs.append(os_[:, :_HID] * (1.0 / (os_[:, _HID:] + 1e-16)))
        g = jnp.concatenate(outs, axis=1) + b_ref[...]
        return jnp.maximum(g, 0.0)

    def adj_wait():
        adj_cp.wait()
        return adj_vmem[...]

    g1 = gat(x_ref[...], w1_ref, as1_ref, ad1_ref, b1_ref, adj_wait)
    g2 = gat(g1, w2_ref, as2_ref, ad2_ref, b2_ref, lambda: adj_vmem[...])

    t = jnp.maximum(
        jnp.dot(g2, fm1_ref[...], preferred_element_type=jnp.float32)
        + fmb1_ref[...], 0.0)
    mean = jnp.dot(t, fm2_ref[...], preferred_element_type=jnp.float32) + fmb2_ref[...]
    t2 = jnp.maximum(
        jnp.dot(g2, fl1_ref[...], preferred_element_type=jnp.float32)
        + flb1_ref[...], 0.0)
    logvar = jnp.dot(t2, fl2_ref[...], preferred_element_type=jnp.float32) + flb2_ref[...]
    std = jnp.exp(0.5 * logvar)
    z = mean + eps_ref[...] * std
    zt = jnp.maximum(
        jnp.dot(z, atw_ref[...], preferred_element_type=jnp.float32)
        + atb_ref[...], 0.0)
    u = jax.lax.dot_general(
        zt, dw_ref[0:1, :], (((1,), (1,)), ((), ())),
        preferred_element_type=jnp.float32)
    vT = jax.lax.dot_general(
        dw_ref[1:2, :], zt, (((1,), (1,)), ((), ())),
        preferred_element_type=jnp.float32)
    r = jax.nn.sigmoid(u + vT + db_ref[0, 0])
    rows = jax.lax.broadcasted_iota(jnp.int32, (_N, _N), 0)
    colz = jax.lax.broadcasted_iota(jnp.int32, (_N, _N), 1)
    rec_ref[...] = jnp.where(rows == colz, 0.0, r)
    mean_ref[...] = mean
    logvar_ref[...] = logvar


_EPS_CACHE = []


def _eps_const():
    if _EPS_CACHE:
        return _EPS_CACHE[0]
    n = _N * _LAT
    x0 = np.zeros(n, dtype=np.uint32)
    x1 = np.arange(n, dtype=np.uint32)
    k0, k1 = np.uint32(0), np.uint32(42)
    ks = [k0, k1, np.uint32(k0 ^ k1 ^ np.uint32(0x1BD11BDA))]
    rot = [[13, 15, 26, 6], [17, 29, 16, 24]]

    def rotl(v, d):
        return ((v << np.uint32(d)) | (v >> np.uint32(32 - d))).astype(np.uint32)

    x0 = (x0 + ks[0]).astype(np.uint32)
    x1 = (x1 + ks[1]).astype(np.uint32)
    for d in range(5):
        for r in rot[d % 2]:
            x0 = (x0 + x1).astype(np.uint32)
            x1 = rotl(x1, r) ^ x0
        x0 = (x0 + ks[(d + 1) % 3]).astype(np.uint32)
        x1 = (x1 + ks[(d + 2) % 3] + np.uint32(d + 1)).astype(np.uint32)
    bits = x0 ^ x1
    fb = ((bits >> np.uint32(9)) | np.uint32(0x3F800000)).view(np.float32)
    u01 = fb - np.float32(1.0)
    lo = np.float32(np.nextafter(np.float32(-1.0), np.float32(0.0)))
    u = np.maximum(lo, (u01 * (np.float32(1.0) - lo) + lo).astype(np.float32))
    w = (-np.log((np.float32(1.0) - u) * (np.float32(1.0) + u))).astype(np.float32)
    ws = (w - np.float32(2.5)).astype(np.float32)
    p1 = np.float32(2.81022636e-08)
    for c in (3.43273939e-07, -3.5233877e-06, -4.39150654e-06, 0.00021858087,
              -0.00125372503, -0.00417768164, 0.246640727, 1.50140941):
        p1 = (np.float32(c) + p1 * ws).astype(np.float32)
    wb = (np.sqrt(w) - np.float32(3.0)).astype(np.float32)
    p2 = np.float32(-0.000200214257)
    for c in (0.000100950558, 0.00134934322, -0.00367342844, 0.00573950773,
              -0.0076224613, 0.00943887047, 1.00167406, 2.83297682):
        p2 = (np.float32(c) + p2 * wb).astype(np.float32)
    p = np.where(w < np.float32(5.0), p1, p2).astype(np.float32)
    eps = (np.float32(np.sqrt(np.float32(2.0))) * p * u).astype(np.float32)
    _EPS_CACHE.append(eps.reshape(_N, _LAT))
    return _EPS_CACHE[0]


def kernel(x, adj, W1, att_src1, att_dst1, b1, W2, att_src2, att_dst2, b2,
           fcm_w1, fcm_b1, fcm_w2, fcm_b2, fcl_w1, fcl_b1, fcl_w2, fcl_b2,
           at_w, at_b, dec_w, dec_b):
    eps = jnp.asarray(_eps_const())
    n_in = 23
    in_specs = [pl.BlockSpec(memory_space=pl.ANY) if i == 1
                else pl.BlockSpec() for i in range(n_in)]
    adj_recon, mean, logvar = pl.pallas_call(
        _fused_kernel,
        in_specs=in_specs,
        scratch_shapes=[
            pltpu.MemorySpace.VMEM((_N, _N), jnp.float32),
            pltpu.SemaphoreType.DMA,
        ],
        out_shape=(
            jax.ShapeDtypeStruct((_N, _N), jnp.float32),
            jax.ShapeDtypeStruct((_N, _LAT), jnp.float32),
            jax.ShapeDtypeStruct((_N, _LAT), jnp.float32),
        ),
    )(x, adj,
      W1, att_src1, att_dst1, b1.reshape(1, _H4),
      W2, att_src2, att_dst2, b2.reshape(1, _H4),
      fcm_w1, fcm_b1.reshape(1, _LAT), fcm_w2, fcm_b2.reshape(1, _LAT),
      fcl_w1, fcl_b1.reshape(1, _LAT), fcl_w2, fcl_b2.reshape(1, _LAT),
      at_w, at_b.reshape(1, _HID), eps,
      dec_w.reshape(2, _HID), dec_b.reshape(1, 1))
    return adj_recon, mean, logvar


# speedup vs baseline: 1.1973x; 1.1973x over previous
import jax
import jax.numpy as jnp
import numpy as np
from jax.experimental import pallas as pl
from jax.experimental.pallas import tpu as pltpu

_N = 1024
_IN = 256
_HID = 64
_HEADS = 4
_H4 = _HEADS * _HID
_LAT = 64


def _fused_kernel(x_ref, adj_ref,
                  w1_ref, as1_ref, ad1_ref, b1_ref,
                  w2_ref, as2_ref, ad2_ref, b2_ref,
                  fm1_ref, fmb1_ref, fm2_ref, fmb2_ref,
                  fl1_ref, flb1_ref, fl2_ref, flb2_ref,
                  atw_ref, atb_ref, eps_ref, dw_ref, db_ref,
                  rec_ref, mean_ref, logvar_ref,
                  adj_vmem, adj_sem):
    adj_cp = pltpu.make_async_copy(adj_ref, adj_vmem, adj_sem)
    adj_cp.start()
    _HX = _HID + 1
    lane = jax.lax.broadcasted_iota(jnp.int32, (1, _HEADS * _HX), 1)
    ones_row = jnp.where(lane % _HX == _HID, 1.0, 0.0).astype(jnp.float32)
    zc = jnp.zeros((_H4, 1), dtype=jnp.float32)

    def gat(inp, w_ref, asw_ref, adw_ref, b_ref, get_adj):
        w = w_ref[...]
        w_aug = jnp.concatenate(
            [t for k in range(_HEADS)
             for t in (w[:, _HID * k : _HID * (k + 1)], zc)], axis=1)
        h_aug = (jnp.dot(inp, w_aug, preferred_element_type=jnp.float32)
                 + ones_row)
        vecs = []
        for k in range(_HEADS):
            hk = h_aug[:, _HX * k : _HX * k + _HID]
            hx = h_aug[:, _HX * k : _HX * (k + 1)]
            a = jax.lax.dot_general(
                hk, asw_ref[k : k + 1, :], (((1,), (1,)), ((), ())),
                preferred_element_type=jnp.float32)
            bT = jax.lax.dot_general(
                adw_ref[k : k + 1, :], hk, (((1,), (1,)), ((), ())),
                preferred_element_type=jnp.float32)
            c1 = jnp.max(a)
            c2 = jnp.max(bT)
            half = 0.4 * (c1 + c2)
            ea = jnp.exp(a - c1)
            ebT = jnp.exp(bT - c2)
            ea2 = jnp.exp(0.2 * a - (0.2 * c1 + half))
            eb2T = jnp.exp(0.2 * bT - (0.2 * c2 + half))
            vecs.append((hx, ea, ebT, ea2, eb2T))
        adjv = get_adj()
        outs = []
        for hx, ea, ebT, ea2, eb2T in vecs:
            p = adjv * jnp.maximum(ea * ebT, ea2 * eb2T)
            os_ = jax.lax.dot_general(
                p, hx, (((0,), (0,)), ((), ())),
                preferred_element_type=jnp.float32)
            outs.append(os_[:, :_HID] * (1.0 / (os_[:, _HID:] + 1e-16)))
        g = jnp.concatenate(outs, axis=1) + b_ref[...]
        return jnp.maximum(g, 0.0)

    def adj_wait():
        adj_cp.wait()
        return adj_vmem[...]

    g1 = gat(x_ref[...], w1_ref, as1_ref, ad1_ref, b1_ref, adj_wait)
    g2 = gat(g1, w2_ref, as2_ref, ad2_ref, b2_ref, lambda: adj_vmem[...])

    t = jnp.maximum(
        jnp.dot(g2, fm1_ref[...], preferred_element_type=jnp.float32)
        + fmb1_ref[...], 0.0)
    mean = jnp.dot(t, fm2_ref[...], preferred_element_type=jnp.float32) + fmb2_ref[...]
    t2 = jnp.maximum(
        jnp.dot(g2, fl1_ref[...], preferred_element_type=jnp.float32)
        + flb1_ref[...], 0.0)
    logvar = jnp.dot(t2, fl2_ref[...], preferred_element_type=jnp.float32) + flb2_ref[...]
    std = jnp.exp(0.5 * logvar)
    z = mean + eps_ref[...] * std
    zt = jnp.maximum(
        jnp.dot(z, atw_ref[...], preferred_element_type=jnp.float32)
        + atb_ref[...], 0.0)
    u = jax.lax.dot_general(
        zt, dw_ref[0:1, :], (((1,), (1,)), ((), ())),
        preferred_element_type=jnp.float32)
    vT = jax.lax.dot_general(
        dw_ref[1:2, :], zt, (((1,), (1,)), ((), ())),
        preferred_element_type=jnp.float32)
    r = jax.nn.sigmoid(u + vT + db_ref[0, 0])
    rows = jax.lax.broadcasted_iota(jnp.int32, (_N, _N), 0)
    colz = jax.lax.broadcasted_iota(jnp.int32, (_N, _N), 1)
    rec_ref[...] = jnp.where(rows == colz, 0.0, r)
    mean_ref[...] = mean
    logvar_ref[...] = logvar


_EPS_CACHE = []


def _eps_const():
    if _EPS_CACHE:
        return _EPS_CACHE[0]
    n = _N * _LAT
    x0 = np.zeros(n, dtype=np.uint32)
    x1 = np.arange(n, dtype=np.uint32)
    k0, k1 = np.uint32(0), np.uint32(42)
    ks = [k0, k1, np.uint32(k0 ^ k1 ^ np.uint32(0x1BD11BDA))]
    rot = [[13, 15, 26, 6], [17, 29, 16, 24]]

    def rotl(v, d):
        return ((v << np.uint32(d)) | (v >> np.uint32(32 - d))).astype(np.uint32)

    x0 = (x0 + ks[0]).astype(np.uint32)
    x1 = (x1 + ks[1]).astype(np.uint32)
    for d in range(5):
        for r in rot[d % 2]:
            x0 = (x0 + x1).astype(np.uint32)
            x1 = rotl(x1, r) ^ x0
        x0 = (x0 + ks[(d + 1) % 3]).astype(np.uint32)
        x1 = (x1 + ks[(d + 2) % 3] + np.uint32(d + 1)).astype(np.uint32)
    bits = x0 ^ x1
    fb = ((bits >> np.uint32(9)) | np.uint32(0x3F800000)).view(np.float32)
    u01 = fb - np.float32(1.0)
    lo = np.float32(np.nextafter(np.float32(-1.0), np.float32(0.0)))
    u = np.maximum(lo, (u01 * (np.float32(1.0) - lo) + lo).astype(np.float32))
    w = (-np.log((np.float32(1.0) - u) * (np.float32(1.0) + u))).astype(np.float32)
    ws = (w - np.float32(2.5)).astype(np.float32)
    p1 = np.float32(2.81022636e-08)
    for c in (3.43273939e-07, -3.5233877e-06, -4.39150654e-06, 0.00021858087,
              -0.00125372503, -0.00417768164, 0.246640727, 1.50140941):
        p1 = (np.float32(c) + p1 * ws).astype(np.float32)
    wb = (np.sqrt(w) - np.float32(3.0)).astype(np.float32)
    p2 = np.float32(-0.000200214257)
    for c in (0.000100950558, 0.00134934322, -0.00367342844, 0.00573950773,
              -0.0076224613, 0.00943887047, 1.00167406, 2.83297682):
        p2 = (np.float32(c) + p2 * wb).astype(np.float32)
    p = np.where(w < np.float32(5.0), p1, p2).astype(np.float32)
    eps = (np.float32(np.sqrt(np.float32(2.0))) * p * u).astype(np.float32)
    _EPS_CACHE.append(eps.reshape(_N, _LAT))
    return _EPS_CACHE[0]


def kernel(x, adj, W1, att_src1, att_dst1, b1, W2, att_src2, att_dst2, b2,
           fcm_w1, fcm_b1, fcm_w2, fcm_b2, fcl_w1, fcl_b1, fcl_w2, fcl_b2,
           at_w, at_b, dec_w, dec_b):
    eps = jnp.asarray(_eps_const())
    n_in = 23
    in_specs = [pl.BlockSpec(memory_space=pl.ANY) if i == 1
                else pl.BlockSpec() for i in range(n_in)]
    adj_recon, mean, logvar = pl.pallas_call(
        _fused_kernel,
        in_specs=in_specs,
        scratch_shapes=[
            pltpu.MemorySpace.VMEM((_N, _N), jnp.float32),
            pltpu.SemaphoreType.DMA,
        ],
        out_shape=(
            jax.ShapeDtypeStruct((_N, _N), jnp.float32),
            jax.ShapeDtypeStruct((_N, _LAT), jnp.float32),
            jax.ShapeDtypeStruct((_N, _LAT), jnp.float32),
        ),
    )(x, adj,
      W1, att_src1, att_dst1, b1.reshape(1, _H4),
      W2, att_src2, att_dst2, b2.reshape(1, _H4),
      fcm_w1, fcm_b1.reshape(1, _LAT), fcm_w2, fcm_b2.reshape(1, _LAT),
      fcl_w1, fcl_b1.reshape(1, _LAT), fcl_w2, fcl_b2.reshape(1, _LAT),
      at_w, at_b.reshape(1, _HID), eps,
      dec_w.reshape(2, _HID), dec_b.reshape(1, 1))
    return adj_recon, mean, logvar
